# baseline (speedup 1.0000x reference)
import numpy as np
import jax
import jax.numpy as jnp
from jax.experimental import pallas as pl
from jax.experimental.pallas import tpu as pltpu

_B = 32
_L = 2048
_D = 512
_S = 16
_TEMP = 0.15
_NB = 4

_PERM_TABLE = np.array([
    [7, 4, 16, 2, 5, 3, 6, 10, 11, 15, 8, 9, 13, 14, 0, 1],
    [7, 4, 16, 2, 5, 3, 6, 10, 11, 15, 8, 9, 13, 14, 17, 0],
    [7, 4, 16, 2, 5, 3, 6, 18, 10, 11, 15, 8, 9, 13, 14, 17],
    [7, 4, 16, 19, 2, 5, 3, 6, 18, 10, 11, 15, 8, 9, 13, 14],
    [7, 4, 16, 19, 2, 5, 3, 6, 18, 10, 11, 15, 20, 8, 9, 13],
    [7, 4, 16, 19, 2, 5, 3, 6, 18, 10, 11, 15, 20, 8, 9, 13],
    [7, 4, 16, 19, 2, 5, 3, 22, 6, 18, 10, 11, 15, 20, 8, 9],
    [7, 4, 16, 19, 2, 5, 3, 22, 6, 18, 10, 11, 15, 20, 8, 9],
    [7, 4, 16, 19, 2, 5, 3, 22, 6, 18, 10, 11, 15, 20, 8, 24],
    [7, 4, 16, 19, 2, 5, 3, 22, 6, 18, 10, 11, 15, 20, 8, 24],
    [7, 4, 16, 19, 2, 5, 3, 22, 6, 18, 10, 11, 15, 20, 8, 24],
    [7, 4, 16, 19, 2, 5, 3, 22, 6, 18, 10, 11, 15, 20, 8, 24],
    [7, 4, 16, 19, 2, 5, 3, 22, 6, 18, 10, 11, 15, 20, 8, 24],
    [7, 4, 29, 16, 19, 2, 5, 3, 22, 6, 18, 10, 11, 15, 20, 8],
    [7, 4, 29, 16, 19, 2, 5, 30, 3, 22, 6, 18, 10, 11, 15, 20],
    [31, 7, 4, 29, 16, 19, 2, 5, 30, 3, 22, 6, 18, 10, 11, 15],
], dtype=np.float32)
_PERM_TABLE_T = np.ascontiguousarray(_PERM_TABLE.T)


def _iota(shape, dim):
    return jax.lax.broadcasted_iota(jnp.int32, shape, dim).astype(jnp.float32)


def _tail(ag_s, tot_s, cnt_ag, tab):
    cnt_ab = jnp.float32(_L) - cnt_ag
    ag_emb = ag_s / jnp.maximum(cnt_ag, 1.0)
    ab_emb = (tot_s - ag_s) / jnp.maximum(cnt_ab, 1.0)

    valid = (cnt_ag > 0).astype(jnp.float32)
    n_avail = jnp.sum(valid)

    ii = _iota((_B, _B), 0)
    jj = _iota((_B, _B), 1)
    lower = (jj <= ii).astype(jnp.float32)
    pos = jnp.dot(lower, valid, preferred_element_type=jnp.float32) - 1.0
    scatter = (pos == jj).astype(jnp.float32) * valid
    i_col = _iota((_B, 1), 0)
    idx = jnp.dot(scatter.T, i_col, preferred_element_type=jnp.float32)

    t_col = _iota((_S, 1), 0)
    ksel = (t_col == (n_avail - 17.0)).astype(jnp.float32)
    perm = jnp.dot(tab, ksel, preferred_element_type=jnp.float32)
    selpos = jnp.where(n_avail > _S, perm, t_col)
    n_eff = jnp.minimum(n_avail, jnp.float32(_S))

    jb = _iota((_S, _B), 1)
    onehot_sel = (selpos == jb).astype(jnp.float32)
    fidx = jnp.dot(onehot_sel, idx, preferred_element_type=jnp.float32)
    row_live = (t_col < n_eff).astype(jnp.float32)
    w = (fidx == jb).astype(jnp.float32) * row_live

    ab = jnp.dot(w, ab_emb, preferred_element_type=jnp.float32)
    ag = jnp.dot(w, ag_emb, preferred_element_type=jnp.float32)
    abn = ab / (jnp.sqrt(jnp.sum(ab * ab, axis=1, keepdims=True)) + 1e-8)
    agn = ag / (jnp.sqrt(jnp.sum(ag * ag, axis=1, keepdims=True)) + 1e-8)

    sim = jax.lax.dot_general(
        abn, agn, dimension_numbers=(((1,), (1,)), ((), ())),
        preferred_element_type=jnp.float32) / _TEMP

    tt = _iota((_S, _S), 0)
    cc = _iota((_S, _S), 1)
    col_live = (cc < n_eff).astype(jnp.float32)
    simm = jnp.where(col_live > 0, sim, -1e30)
    mx = jnp.max(simm, axis=1, keepdims=True)
    lse = jnp.log(jnp.sum(jnp.exp(simm - mx) * col_live, axis=1,
                          keepdims=True)) + mx
    diag = jnp.sum(sim * (tt == cc).astype(jnp.float32) * col_live,
                   axis=1, keepdims=True)
    logp = (diag - lse) * row_live
    return -jnp.sum(logp, axis=0, keepdims=True)[:, 0:1] / n_eff


def _fused_body(ct_ref, tab_ref, sa_ref, sb_ref, out_ref,
                ag_scr, tot_scr, cnt_scr):
    b = pl.program_id(0)
    h = _L // 2
    ones = jnp.ones((1, h), jnp.float32)
    for i in range(_NB):
        row = _NB * b + i
        m = (ct_ref[pl.ds(row, 1), :] == 0).astype(jnp.float32)
        ra = jnp.concatenate([m[:, :h], ones], axis=0)
        rb = jnp.concatenate([m[:, h:], ones], axis=0)
        r = (jnp.dot(ra, sa_ref[i], preferred_element_type=jnp.float32)
             + jnp.dot(rb, sb_ref[i], preferred_element_type=jnp.float32))
        ag_scr[pl.ds(row, 1), :] = r[0:1]
        tot_scr[pl.ds(row, 1), :] = r[1:2]
        cnt_scr[pl.ds(row, 1), :] = jnp.broadcast_to(
            jnp.sum(m, axis=1, keepdims=True), (1, 128))

    @pl.when(b == (_B // _NB) - 1)
    def _():
        out_ref[...] = _tail(ag_scr[...], tot_scr[...],
                             cnt_scr[:, 0:1], tab_ref[...])


def kernel(s_i, chain_type):
    loss = pl.pallas_call(
        _fused_body,
        grid=(_B // _NB,),
        in_specs=[
            pl.BlockSpec((_B, _L), lambda b: (0, 0)),
            pl.BlockSpec((_S, _S), lambda b: (0, 0)),
            pl.BlockSpec((_NB, _L // 2, _D), lambda b: (b, 0, 0)),
            pl.BlockSpec((_NB, _L // 2, _D), lambda b: (b, 1, 0)),
        ],
        out_specs=pl.BlockSpec((1, 1), lambda b: (0, 0)),
        out_shape=jax.ShapeDtypeStruct((1, 1), jnp.float32),
        scratch_shapes=[
            pltpu.VMEM((_B, _D), jnp.float32),
            pltpu.VMEM((_B, _D), jnp.float32),
            pltpu.VMEM((_B, 128), jnp.float32),
        ],
    )(chain_type.astype(jnp.int32), jnp.asarray(_PERM_TABLE_T), s_i, s_i)
    return loss[0, 0]


# speedup vs baseline: 1.0225x; 1.0225x over previous
import numpy as np
import jax
import jax.numpy as jnp
from jax.experimental import pallas as pl
from jax.experimental.pallas import tpu as pltpu

_B = 32
_L = 2048
_D = 512
_S = 16
_TEMP = 0.15
_NB = 2

_PERM_TABLE = np.array([
    [7, 4, 16, 2, 5, 3, 6, 10, 11, 15, 8, 9, 13, 14, 0, 1],
    [7, 4, 16, 2, 5, 3, 6, 10, 11, 15, 8, 9, 13, 14, 17, 0],
    [7, 4, 16, 2, 5, 3, 6, 18, 10, 11, 15, 8, 9, 13, 14, 17],
    [7, 4, 16, 19, 2, 5, 3, 6, 18, 10, 11, 15, 8, 9, 13, 14],
    [7, 4, 16, 19, 2, 5, 3, 6, 18, 10, 11, 15, 20, 8, 9, 13],
    [7, 4, 16, 19, 2, 5, 3, 6, 18, 10, 11, 15, 20, 8, 9, 13],
    [7, 4, 16, 19, 2, 5, 3, 22, 6, 18, 10, 11, 15, 20, 8, 9],
    [7, 4, 16, 19, 2, 5, 3, 22, 6, 18, 10, 11, 15, 20, 8, 9],
    [7, 4, 16, 19, 2, 5, 3, 22, 6, 18, 10, 11, 15, 20, 8, 24],
    [7, 4, 16, 19, 2, 5, 3, 22, 6, 18, 10, 11, 15, 20, 8, 24],
    [7, 4, 16, 19, 2, 5, 3, 22, 6, 18, 10, 11, 15, 20, 8, 24],
    [7, 4, 16, 19, 2, 5, 3, 22, 6, 18, 10, 11, 15, 20, 8, 24],
    [7, 4, 16, 19, 2, 5, 3, 22, 6, 18, 10, 11, 15, 20, 8, 24],
    [7, 4, 29, 16, 19, 2, 5, 3, 22, 6, 18, 10, 11, 15, 20, 8],
    [7, 4, 29, 16, 19, 2, 5, 30, 3, 22, 6, 18, 10, 11, 15, 20],
    [31, 7, 4, 29, 16, 19, 2, 5, 30, 3, 22, 6, 18, 10, 11, 15],
], dtype=np.float32)
_PERM_TABLE_T = np.ascontiguousarray(_PERM_TABLE.T)


def _iota(shape, dim):
    return jax.lax.broadcasted_iota(jnp.int32, shape, dim).astype(jnp.float32)


def _tail(ag_s, tot_s, cnt_ag, tab):
    cnt_ab = jnp.float32(_L) - cnt_ag
    ag_emb = ag_s / jnp.maximum(cnt_ag, 1.0)
    ab_emb = (tot_s - ag_s) / jnp.maximum(cnt_ab, 1.0)

    valid = (cnt_ag > 0).astype(jnp.float32)
    n_avail = jnp.sum(valid)

    ii = _iota((_B, _B), 0)
    jj = _iota((_B, _B), 1)
    lower = (jj <= ii).astype(jnp.float32)
    pos = jnp.dot(lower, valid, preferred_element_type=jnp.float32) - 1.0
    scatter = (pos == jj).astype(jnp.float32) * valid
    i_col = _iota((_B, 1), 0)
    idx = jnp.dot(scatter.T, i_col, preferred_element_type=jnp.float32)

    t_col = _iota((_S, 1), 0)
    ksel = (t_col == (n_avail - 17.0)).astype(jnp.float32)
    perm = jnp.dot(tab, ksel, preferred_element_type=jnp.float32)
    selpos = jnp.where(n_avail > _S, perm, t_col)
    n_eff = jnp.minimum(n_avail, jnp.float32(_S))

    jb = _iota((_S, _B), 1)
    onehot_sel = (selpos == jb).astype(jnp.float32)
    fidx = jnp.dot(onehot_sel, idx, preferred_element_type=jnp.float32)
    row_live = (t_col < n_eff).astype(jnp.float32)
    w = (fidx == jb).astype(jnp.float32) * row_live

    ab = jnp.dot(w, ab_emb, preferred_element_type=jnp.float32)
    ag = jnp.dot(w, ag_emb, preferred_element_type=jnp.float32)
    abn = ab / (jnp.sqrt(jnp.sum(ab * ab, axis=1, keepdims=True)) + 1e-8)
    agn = ag / (jnp.sqrt(jnp.sum(ag * ag, axis=1, keepdims=True)) + 1e-8)

    sim = jax.lax.dot_general(
        abn, agn, dimension_numbers=(((1,), (1,)), ((), ())),
        preferred_element_type=jnp.float32) / _TEMP

    tt = _iota((_S, _S), 0)
    cc = _iota((_S, _S), 1)
    col_live = (cc < n_eff).astype(jnp.float32)
    simm = jnp.where(col_live > 0, sim, -1e30)
    mx = jnp.max(simm, axis=1, keepdims=True)
    lse = jnp.log(jnp.sum(jnp.exp(simm - mx) * col_live, axis=1,
                          keepdims=True)) + mx
    diag = jnp.sum(sim * (tt == cc).astype(jnp.float32) * col_live,
                   axis=1, keepdims=True)
    logp = (diag - lse) * row_live
    return -jnp.sum(logp, axis=0, keepdims=True)[:, 0:1] / n_eff


def _fused_body(ct_ref, tab_ref, s0_ref, s1_ref, s2_ref, s3_ref, out_ref,
                ag_scr, tot_scr, cnt_scr):
    b = pl.program_id(0)
    h = _L // 4
    ones = jnp.ones((1, h), jnp.float32)
    srefs = (s0_ref, s1_ref, s2_ref, s3_ref)
    for i in range(_NB):
        row = _NB * b + i
        m = (ct_ref[pl.ds(row, 1), :] == 0).astype(jnp.float32)
        r = None
        for q in range(4):
            rq = jnp.concatenate([m[:, q * h:(q + 1) * h], ones], axis=0)
            d = jnp.dot(rq, srefs[q][i], preferred_element_type=jnp.float32)
            r = d if r is None else r + d
        ag_scr[pl.ds(row, 1), :] = r[0:1]
        tot_scr[pl.ds(row, 1), :] = r[1:2]
        cnt_scr[pl.ds(row, 1), :] = jnp.broadcast_to(
            jnp.sum(m, axis=1, keepdims=True), (1, 128))

    @pl.when(b == (_B // _NB) - 1)
    def _():
        out_ref[...] = _tail(ag_scr[...], tot_scr[...],
                             cnt_scr[:, 0:1], tab_ref[...])


def kernel(s_i, chain_type):
    loss = pl.pallas_call(
        _fused_body,
        grid=(_B // _NB,),
        in_specs=[
            pl.BlockSpec((_B, _L), lambda b: (0, 0)),
            pl.BlockSpec((_S, _S), lambda b: (0, 0)),
            pl.BlockSpec((_NB, _L // 4, _D), lambda b: (b, 0, 0)),
            pl.BlockSpec((_NB, _L // 4, _D), lambda b: (b, 1, 0)),
            pl.BlockSpec((_NB, _L // 4, _D), lambda b: (b, 2, 0)),
            pl.BlockSpec((_NB, _L // 4, _D), lambda b: (b, 3, 0)),
        ],
        out_specs=pl.BlockSpec((1, 1), lambda b: (0, 0)),
        out_shape=jax.ShapeDtypeStruct((1, 1), jnp.float32),
        scratch_shapes=[
            pltpu.VMEM((_B, _D), jnp.float32),
            pltpu.VMEM((_B, _D), jnp.float32),
            pltpu.VMEM((_B, 128), jnp.float32),
        ],
    )(chain_type.astype(jnp.int32), jnp.asarray(_PERM_TABLE_T), s_i, s_i, s_i, s_i)
    return loss[0, 0]
